# baseline (speedup 1.0000x reference)
import functools

import jax
import jax.numpy as jnp
from jax import lax
from jax.experimental import pallas as pl
from jax.experimental.pallas import tpu as pltpu
from jax.experimental.pallas import tpu_sc as plsc

N = 10000
E = 320000
D = 128

NC = 2
NS = 16
NW = NC * NS
K = 80
RPT = 624
TAIL = N - NS * RPT


def _make_pipeline(dst_hbm, idx_s, acc, bufs, ebase, nchunk, k=K):
  idx_da, idx_db, rows_a, rows_b, sem_ga, sem_gb = bufs

  def issue_g(i, rows_buf, idx_buf, sem, table):
    pltpu.async_copy(table.at[idx_s.at[pl.ds(i * k, k)]], rows_buf, sem)
    pltpu.async_copy(dst_hbm.at[pl.ds(ebase + i * k, k)], idx_buf, sem)

  def wait_g(rows_buf, idx_buf, sem, table):
    pltpu.make_async_copy(table.at[idx_s.at[pl.ds(0, k)]],
                          rows_buf, sem).wait()
    pltpu.make_async_copy(dst_hbm.at[pl.ds(0, k)], idx_buf, sem).wait()

  def run(table):
    issue_g(0, rows_a, idx_da, sem_ga, table)

    def body(j, carry):
      issue_g(2 * j + 1, rows_b, idx_db, sem_gb, table)
      wait_g(rows_a, idx_da, sem_ga, table)
      pltpu.sync_copy(rows_a, acc.at[idx_da], add=True)
      issue_g(2 * j + 2, rows_a, idx_da, sem_ga, table)
      wait_g(rows_b, idx_db, sem_gb, table)
      pltpu.sync_copy(rows_b, acc.at[idx_db], add=True)
      return carry

    if nchunk % 2 == 1:
      lax.fori_loop(0, (nchunk - 1) // 2, body, 0)
    else:
      lax.fori_loop(0, nchunk // 2 - 1, body, 0)
      issue_g(nchunk - 1, rows_b, idx_db, sem_gb, table)
    wait_g(rows_a, idx_da, sem_ga, table)
    pltpu.sync_copy(rows_a, acc.at[idx_da], add=True)
    if nchunk % 2 == 0:
      wait_g(rows_b, idx_db, sem_gb, table)
      pltpu.sync_copy(rows_b, acc.at[idx_db], add=True)

  return run


_SC_MESH = plsc.VectorSubcoreMesh(core_axis_name="c", subcore_axis_name="s")


def _sc_scratch(idx_len):
  return [
      pltpu.VMEM((idx_len,), jnp.int32),
      pltpu.VMEM((K,), jnp.int32),
      pltpu.VMEM((K,), jnp.int32),
      pltpu.VMEM((K, D), jnp.float32),
      pltpu.VMEM((K, D), jnp.float32),
      pltpu.VMEM_SHARED((N, D), jnp.float32),
      pltpu.SemaphoreType.DMA,
      pltpu.SemaphoreType.DMA,
  ]


def _zero_acc(zeros_hbm, acc, row0, s):
  pltpu.sync_copy(zeros_hbm, acc.at[pl.ds(row0, RPT)])

  @pl.when(s == NS - 1)
  def _tail():
    pltpu.sync_copy(zeros_hbm.at[pl.ds(0, TAIL)],
                    acc.at[pl.ds(NS * RPT, TAIL)])


def _dump_acc(acc, out, out_base, row0, s):
  pltpu.sync_copy(acc.at[pl.ds(row0, RPT)],
                  out.at[pl.ds(out_base + row0, RPT)])

  @pl.when(s == NS - 1)
  def _tail():
    pltpu.sync_copy(acc.at[pl.ds(NS * RPT, TAIL)],
                    out.at[pl.ds(out_base + NS * RPT, TAIL)])


_EPT2 = E // NS
_NCHUNK2 = _EPT2 // K


@functools.partial(
    pl.kernel,
    out_type=(jax.ShapeDtypeStruct((N, D), jnp.float32),
              jax.ShapeDtypeStruct((N, D), jnp.float32)),
    mesh=_SC_MESH, scratch_types=_sc_scratch(_EPT2))
def _agg2(feat_hbm, state_hbm, src_hbm, dst_hbm, zeros_hbm, p_out, q_out,
          idx_s, idx_da, idx_db, rows_a, rows_b, acc, sem_ga, sem_gb):
  c = lax.axis_index("c")
  s = lax.axis_index("s")
  ebase = s * _EPT2
  row0 = s * RPT

  pltpu.sync_copy(src_hbm.at[pl.ds(ebase, _EPT2)], idx_s)
  _zero_acc(zeros_hbm, acc, row0, s)
  plsc.subcore_barrier()

  run = _make_pipeline(dst_hbm, idx_s, acc,
                       (idx_da, idx_db, rows_a, rows_b, sem_ga, sem_gb),
                       ebase, _NCHUNK2)

  @pl.when(c == 0)
  def _feat():
    run(feat_hbm)

  @pl.when(c == 1)
  def _state():
    run(state_hbm)

  plsc.subcore_barrier()

  @pl.when(c == 0)
  def _dump_p():
    _dump_acc(acc, p_out, 0, row0, s)

  @pl.when(c == 1)
  def _dump_q():
    _dump_acc(acc, q_out, 0, row0, s)


_EPT1 = E // NW
_NCHUNK1 = _EPT1 // K


@functools.partial(
    pl.kernel,
    out_type=jax.ShapeDtypeStruct((NC * N, D), jnp.float32),
    mesh=_SC_MESH, scratch_types=_sc_scratch(_EPT1))
def _agg1(table_hbm, src_hbm, dst_hbm, zeros_hbm, out,
          idx_s, idx_da, idx_db, rows_a, rows_b, acc, sem_ga, sem_gb):
  c = lax.axis_index("c")
  s = lax.axis_index("s")
  ebase = (c * NS + s) * _EPT1
  row0 = s * RPT

  pltpu.sync_copy(src_hbm.at[pl.ds(ebase, _EPT1)], idx_s)
  _zero_acc(zeros_hbm, acc, row0, s)
  plsc.subcore_barrier()

  _make_pipeline(dst_hbm, idx_s, acc,
                 (idx_da, idx_db, rows_a, rows_b, sem_ga, sem_gb),
                 ebase, _NCHUNK1)(table_hbm)

  plsc.subcore_barrier()
  _dump_acc(acc, out, c * N, row0, s)


_R = 2000


def _row_spec(off_blocks, width=D):
  return pl.BlockSpec((_R, width), lambda i, o=off_blocks: (o + i, 0))


def _full_spec(shape):
  return pl.BlockSpec(shape, lambda i: (0, 0))


def _gate1_body(p, q, f, st, wnt, wnb, wst, wsb, b, z_o, rs_o):
  acc = jnp.dot(p[...], wnt[...], preferred_element_type=jnp.float32)
  acc += jnp.dot(q[...], wnb[...], preferred_element_type=jnp.float32)
  acc += jnp.dot(f[...], wst[...], preferred_element_type=jnp.float32)
  acc += jnp.dot(st[...], wsb[...], preferred_element_type=jnp.float32)
  zr = jax.nn.sigmoid(acc + b[...])
  z_o[...] = zr[:, :D]
  rs_o[...] = zr[:, D:] * st[...]


def _half_spec(half, width):
  return pl.BlockSpec((D, width), lambda i, h=half: (h, 0))


def _gate1(p, q, feat, state, w_nbr, w_self, b):
  in_specs = [
      _row_spec(0), _row_spec(0),
      _row_spec(0), _row_spec(0),
      _half_spec(0, 2 * D), _half_spec(1, 2 * D),
      _half_spec(0, 2 * D), _half_spec(1, 2 * D),
      _full_spec((1, 2 * D)),
  ]
  return pl.pallas_call(
      _gate1_body, grid=(N // _R,), in_specs=in_specs,
      out_specs=[_row_spec(0)] * 2,
      out_shape=[jax.ShapeDtypeStruct((N, D), jnp.float32)] * 2,
  )(p, q, feat, state, w_nbr, w_nbr, w_self, w_self, b)


def _gate2_body(p, s0, s1, f, rs, z, st, wnt, wnb, wst, wsb, b, out):
  acc = jnp.dot(p[...], wnt[...], preferred_element_type=jnp.float32)
  acc += jnp.dot(s0[...] + s1[...], wnb[...],
                 preferred_element_type=jnp.float32)
  acc += jnp.dot(f[...], wst[...], preferred_element_type=jnp.float32)
  acc += jnp.dot(rs[...], wsb[...], preferred_element_type=jnp.float32)
  c = jnp.tanh(acc + b[...])
  zz = z[...]
  out[...] = zz * st[...] + (1.0 - zz) * c


def _gate2(p, s2, feat, rs, z, state, w_nbr, w_self, b):
  nb = N // _R
  in_specs = [
      _row_spec(0),
      _row_spec(0), _row_spec(nb),
      _row_spec(0), _row_spec(0), _row_spec(0), _row_spec(0),
      _half_spec(0, D), _half_spec(1, D),
      _half_spec(0, D), _half_spec(1, D),
      _full_spec((1, D)),
  ]
  return pl.pallas_call(
      _gate2_body, grid=(nb,), in_specs=in_specs,
      out_specs=_row_spec(0),
      out_shape=jax.ShapeDtypeStruct((N, D), jnp.float32),
  )(p, s2, s2, feat, rs, z, state, w_nbr, w_nbr, w_self, w_self, b)


def kernel(feat, state, edge_index, W_zr_nbr, W_zr_self, b_zr,
           W_c_nbr, W_c_self, b_c):
  src = edge_index[0].astype(jnp.int32)
  dst = edge_index[1].astype(jnp.int32)
  zeros = jnp.zeros((RPT, D), jnp.float32)

  p, q = _agg2(feat, state, src, dst, zeros)
  z, rs = _gate1(p, q, feat, state, W_zr_nbr, W_zr_self,
                 b_zr.reshape(1, 2 * D))
  s2 = _agg1(rs, src, dst, zeros)
  return _gate2(p, s2, feat, rs, z, state, W_c_nbr, W_c_self,
                b_c.reshape(1, D))


# speedup vs baseline: 1.0592x; 1.0592x over previous
import functools

import jax
import jax.numpy as jnp
from jax import lax
from jax.experimental import pallas as pl
from jax.experimental.pallas import tpu as pltpu
from jax.experimental.pallas import tpu_sc as plsc

N = 10000
E = 320000
D = 128

NC = 2
NS = 16
NW = NC * NS
K = 80
RPT = 624
TAIL = N - NS * RPT


def _make_pipeline(dst_hbm, idx_s, acc, bufs, ebase, nchunk, k=K):
  idx_da, idx_db, rows_a, rows_b, sem_ga, sem_gb = bufs

  def issue_g(i, rows_buf, idx_buf, sem, table):
    pltpu.async_copy(table.at[idx_s.at[pl.ds(i * k, k)]], rows_buf, sem)
    pltpu.async_copy(dst_hbm.at[pl.ds(ebase + i * k, k)], idx_buf, sem)

  def wait_g(rows_buf, idx_buf, sem, table):
    pltpu.make_async_copy(table.at[idx_s.at[pl.ds(0, k)]],
                          rows_buf, sem).wait()
    pltpu.make_async_copy(dst_hbm.at[pl.ds(0, k)], idx_buf, sem).wait()

  def run(table):
    issue_g(0, rows_a, idx_da, sem_ga, table)

    def body(j, carry):
      issue_g(2 * j + 1, rows_b, idx_db, sem_gb, table)
      wait_g(rows_a, idx_da, sem_ga, table)
      pltpu.sync_copy(rows_a, acc.at[idx_da], add=True)
      issue_g(2 * j + 2, rows_a, idx_da, sem_ga, table)
      wait_g(rows_b, idx_db, sem_gb, table)
      pltpu.sync_copy(rows_b, acc.at[idx_db], add=True)
      return carry

    if nchunk % 2 == 1:
      lax.fori_loop(0, (nchunk - 1) // 2, body, 0)
    else:
      lax.fori_loop(0, nchunk // 2 - 1, body, 0)
      issue_g(nchunk - 1, rows_b, idx_db, sem_gb, table)
    wait_g(rows_a, idx_da, sem_ga, table)
    pltpu.sync_copy(rows_a, acc.at[idx_da], add=True)
    if nchunk % 2 == 0:
      wait_g(rows_b, idx_db, sem_gb, table)
      pltpu.sync_copy(rows_b, acc.at[idx_db], add=True)

  return run


_SC_MESH = plsc.VectorSubcoreMesh(core_axis_name="c", subcore_axis_name="s")


def _sc_scratch(idx_len):
  return [
      pltpu.VMEM((idx_len,), jnp.int32),
      pltpu.VMEM((K,), jnp.int32),
      pltpu.VMEM((K,), jnp.int32),
      pltpu.VMEM((K, D), jnp.float32),
      pltpu.VMEM((K, D), jnp.float32),
      pltpu.VMEM_SHARED((N, D), jnp.float32),
      pltpu.SemaphoreType.DMA,
      pltpu.SemaphoreType.DMA,
  ]


def _zero_acc(zeros_hbm, acc, row0, s):
  pltpu.sync_copy(zeros_hbm, acc.at[pl.ds(row0, RPT)])

  @pl.when(s == NS - 1)
  def _tail():
    pltpu.sync_copy(zeros_hbm.at[pl.ds(0, TAIL)],
                    acc.at[pl.ds(NS * RPT, TAIL)])


def _dump_acc(acc, out, out_base, row0, s):
  pltpu.sync_copy(acc.at[pl.ds(row0, RPT)],
                  out.at[pl.ds(out_base + row0, RPT)])

  @pl.when(s == NS - 1)
  def _tail():
    pltpu.sync_copy(acc.at[pl.ds(NS * RPT, TAIL)],
                    out.at[pl.ds(out_base + NS * RPT, TAIL)])


_EPT2 = E // NS
_K2 = 160
_SUBS = ((0, 8000), (8000, 8000), (16000, 4000))


@functools.partial(
    pl.kernel,
    out_type=(jax.ShapeDtypeStruct((N, D), jnp.float32),
              jax.ShapeDtypeStruct((N, D), jnp.float32)),
    mesh=_SC_MESH,
    scratch_types=[
        pltpu.VMEM((8000,), jnp.int32),
        pltpu.VMEM((_K2,), jnp.int32),
        pltpu.VMEM((_K2,), jnp.int32),
        pltpu.VMEM((_K2, D), jnp.float32),
        pltpu.VMEM((_K2, D), jnp.float32),
        pltpu.VMEM_SHARED((N, D), jnp.float32),
        pltpu.SemaphoreType.DMA,
        pltpu.SemaphoreType.DMA,
    ])
def _agg2(feat_hbm, state_hbm, src_hbm, dst_hbm, zeros_hbm, p_out, q_out,
          idx_s, idx_da, idx_db, rows_a, rows_b, acc, sem_ga, sem_gb):
  c = lax.axis_index("c")
  s = lax.axis_index("s")
  ebase = s * _EPT2
  row0 = s * RPT

  _zero_acc(zeros_hbm, acc, row0, s)
  plsc.subcore_barrier()

  def run(table):
    for off, ln in _SUBS:
      pltpu.sync_copy(src_hbm.at[pl.ds(ebase + off, ln)],
                      idx_s.at[pl.ds(0, ln)])
      _make_pipeline(dst_hbm, idx_s, acc,
                     (idx_da, idx_db, rows_a, rows_b, sem_ga, sem_gb),
                     ebase + off, ln // _K2, k=_K2)(table)

  @pl.when(c == 0)
  def _feat():
    run(feat_hbm)

  @pl.when(c == 1)
  def _state():
    run(state_hbm)

  plsc.subcore_barrier()

  @pl.when(c == 0)
  def _dump_p():
    _dump_acc(acc, p_out, 0, row0, s)

  @pl.when(c == 1)
  def _dump_q():
    _dump_acc(acc, q_out, 0, row0, s)


_EPT1 = E // NW
_NCHUNK1 = _EPT1 // K


@functools.partial(
    pl.kernel,
    out_type=jax.ShapeDtypeStruct((NC * N, D), jnp.float32),
    mesh=_SC_MESH, scratch_types=_sc_scratch(_EPT1))
def _agg1(table_hbm, src_hbm, dst_hbm, zeros_hbm, out,
          idx_s, idx_da, idx_db, rows_a, rows_b, acc, sem_ga, sem_gb):
  c = lax.axis_index("c")
  s = lax.axis_index("s")
  ebase = (c * NS + s) * _EPT1
  row0 = s * RPT

  pltpu.sync_copy(src_hbm.at[pl.ds(ebase, _EPT1)], idx_s)
  _zero_acc(zeros_hbm, acc, row0, s)
  plsc.subcore_barrier()

  _make_pipeline(dst_hbm, idx_s, acc,
                 (idx_da, idx_db, rows_a, rows_b, sem_ga, sem_gb),
                 ebase, _NCHUNK1)(table_hbm)

  plsc.subcore_barrier()
  _dump_acc(acc, out, c * N, row0, s)


_R = 2000


def _row_spec(off_blocks, width=D):
  return pl.BlockSpec((_R, width), lambda i, o=off_blocks: (o + i, 0))


def _full_spec(shape):
  return pl.BlockSpec(shape, lambda i: (0, 0))


def _gate1_body(p, q, f, st, wnt, wnb, wst, wsb, b, z_o, rs_o):
  acc = jnp.dot(p[...], wnt[...], preferred_element_type=jnp.float32)
  acc += jnp.dot(q[...], wnb[...], preferred_element_type=jnp.float32)
  acc += jnp.dot(f[...], wst[...], preferred_element_type=jnp.float32)
  acc += jnp.dot(st[...], wsb[...], preferred_element_type=jnp.float32)
  zr = jax.nn.sigmoid(acc + b[...])
  z_o[...] = zr[:, :D]
  rs_o[...] = zr[:, D:] * st[...]


def _half_spec(half, width):
  return pl.BlockSpec((D, width), lambda i, h=half: (h, 0))


def _gate1(p, q, feat, state, w_nbr, w_self, b):
  in_specs = [
      _row_spec(0), _row_spec(0),
      _row_spec(0), _row_spec(0),
      _half_spec(0, 2 * D), _half_spec(1, 2 * D),
      _half_spec(0, 2 * D), _half_spec(1, 2 * D),
      _full_spec((1, 2 * D)),
  ]
  return pl.pallas_call(
      _gate1_body, grid=(N // _R,), in_specs=in_specs,
      out_specs=[_row_spec(0)] * 2,
      out_shape=[jax.ShapeDtypeStruct((N, D), jnp.float32)] * 2,
  )(p, q, feat, state, w_nbr, w_nbr, w_self, w_self, b)


def _gate2_body(p, s0, s1, f, rs, z, st, wnt, wnb, wst, wsb, b, out):
  acc = jnp.dot(p[...], wnt[...], preferred_element_type=jnp.float32)
  acc += jnp.dot(s0[...] + s1[...], wnb[...],
                 preferred_element_type=jnp.float32)
  acc += jnp.dot(f[...], wst[...], preferred_element_type=jnp.float32)
  acc += jnp.dot(rs[...], wsb[...], preferred_element_type=jnp.float32)
  c = jnp.tanh(acc + b[...])
  zz = z[...]
  out[...] = zz * st[...] + (1.0 - zz) * c


def _gate2(p, s2, feat, rs, z, state, w_nbr, w_self, b):
  nb = N // _R
  in_specs = [
      _row_spec(0),
      _row_spec(0), _row_spec(nb),
      _row_spec(0), _row_spec(0), _row_spec(0), _row_spec(0),
      _half_spec(0, D), _half_spec(1, D),
      _half_spec(0, D), _half_spec(1, D),
      _full_spec((1, D)),
  ]
  return pl.pallas_call(
      _gate2_body, grid=(nb,), in_specs=in_specs,
      out_specs=_row_spec(0),
      out_shape=jax.ShapeDtypeStruct((N, D), jnp.float32),
  )(p, s2, s2, feat, rs, z, state, w_nbr, w_nbr, w_self, w_self, b)


def kernel(feat, state, edge_index, W_zr_nbr, W_zr_self, b_zr,
           W_c_nbr, W_c_self, b_c):
  src = edge_index[0].astype(jnp.int32)
  dst = edge_index[1].astype(jnp.int32)
  zeros = jnp.zeros((RPT, D), jnp.float32)

  p, q = _agg2(feat, state, src, dst, zeros)
  z, rs = _gate1(p, q, feat, state, W_zr_nbr, W_zr_self,
                 b_zr.reshape(1, 2 * D))
  s2 = _agg1(rs, src, dst, zeros)
  return _gate2(p, s2, feat, rs, z, state, W_c_nbr, W_c_self,
                b_c.reshape(1, D))


# speedup vs baseline: 1.0815x; 1.0211x over previous
import functools

import jax
import jax.numpy as jnp
from jax import lax
from jax.experimental import pallas as pl
from jax.experimental.pallas import tpu as pltpu
from jax.experimental.pallas import tpu_sc as plsc

N = 10000
E = 320000
D = 128

NC = 2
NS = 16
NW = NC * NS
RPT = 624
TAIL = N - NS * RPT


def _make_pipeline(dst_hbm, idx_s, acc, bufs, ebase, nchunk, k):
  idx_da, idx_db, rows_a, rows_b, sem_ga, sem_gb = bufs

  def issue_g(i, rows_buf, idx_buf, sem, table):
    pltpu.async_copy(table.at[idx_s.at[pl.ds(i * k, k)]], rows_buf, sem)
    pltpu.async_copy(dst_hbm.at[pl.ds(ebase + i * k, k)], idx_buf, sem)

  def wait_g(rows_buf, idx_buf, sem, table):
    pltpu.make_async_copy(table.at[idx_s.at[pl.ds(0, k)]],
                          rows_buf, sem).wait()
    pltpu.make_async_copy(dst_hbm.at[pl.ds(0, k)], idx_buf, sem).wait()

  def run(table):
    issue_g(0, rows_a, idx_da, sem_ga, table)

    def body(j, carry):
      issue_g(2 * j + 1, rows_b, idx_db, sem_gb, table)
      wait_g(rows_a, idx_da, sem_ga, table)
      pltpu.sync_copy(rows_a, acc.at[idx_da], add=True)
      issue_g(2 * j + 2, rows_a, idx_da, sem_ga, table)
      wait_g(rows_b, idx_db, sem_gb, table)
      pltpu.sync_copy(rows_b, acc.at[idx_db], add=True)
      return carry

    if nchunk % 2 == 1:
      lax.fori_loop(0, (nchunk - 1) // 2, body, 0)
    else:
      lax.fori_loop(0, nchunk // 2 - 1, body, 0)
      issue_g(nchunk - 1, rows_b, idx_db, sem_gb, table)
    wait_g(rows_a, idx_da, sem_ga, table)
    pltpu.sync_copy(rows_a, acc.at[idx_da], add=True)
    if nchunk % 2 == 0:
      wait_g(rows_b, idx_db, sem_gb, table)
      pltpu.sync_copy(rows_b, acc.at[idx_db], add=True)

  return run


_SC_MESH = plsc.VectorSubcoreMesh(core_axis_name="c", subcore_axis_name="s")


def _zero_acc(zeros_hbm, acc, row0, s):
  pltpu.sync_copy(zeros_hbm, acc.at[pl.ds(row0, RPT)])

  @pl.when(s == NS - 1)
  def _tail():
    pltpu.sync_copy(zeros_hbm.at[pl.ds(0, TAIL)],
                    acc.at[pl.ds(NS * RPT, TAIL)])


def _dump_acc(acc, out, out_base, row0, s):
  pltpu.sync_copy(acc.at[pl.ds(row0, RPT)],
                  out.at[pl.ds(out_base + row0, RPT)])

  @pl.when(s == NS - 1)
  def _tail():
    pltpu.sync_copy(acc.at[pl.ds(NS * RPT, TAIL)],
                    out.at[pl.ds(out_base + NS * RPT, TAIL)])


_EPT2 = E // NS
_K2 = 160
_SUBS = ((0, 8000), (8000, 8000), (16000, 4000))


@functools.partial(
    pl.kernel,
    out_type=(jax.ShapeDtypeStruct((N, D), jnp.float32),
              jax.ShapeDtypeStruct((N, D), jnp.float32)),
    mesh=_SC_MESH,
    scratch_types=[
        pltpu.VMEM((8000,), jnp.int32),
        pltpu.VMEM((_K2,), jnp.int32),
        pltpu.VMEM((_K2,), jnp.int32),
        pltpu.VMEM((_K2, D), jnp.float32),
        pltpu.VMEM((_K2, D), jnp.float32),
        pltpu.VMEM_SHARED((N, D), jnp.float32),
        pltpu.SemaphoreType.DMA,
        pltpu.SemaphoreType.DMA,
    ])
def _agg2(feat_hbm, state_hbm, src_hbm, dst_hbm, zeros_hbm, p_out, q_out,
          idx_s, idx_da, idx_db, rows_a, rows_b, acc, sem_ga, sem_gb):
  c = lax.axis_index("c")
  s = lax.axis_index("s")
  ebase = s * _EPT2
  row0 = s * RPT

  _zero_acc(zeros_hbm, acc, row0, s)
  plsc.subcore_barrier()

  def run(table):
    for off, ln in _SUBS:
      pltpu.sync_copy(src_hbm.at[pl.ds(ebase + off, ln)],
                      idx_s.at[pl.ds(0, ln)])
      _make_pipeline(dst_hbm, idx_s, acc,
                     (idx_da, idx_db, rows_a, rows_b, sem_ga, sem_gb),
                     ebase + off, ln // _K2, k=_K2)(table)

  @pl.when(c == 0)
  def _feat():
    run(feat_hbm)

  @pl.when(c == 1)
  def _state():
    run(state_hbm)

  plsc.subcore_barrier()

  @pl.when(c == 0)
  def _dump_p():
    _dump_acc(acc, p_out, 0, row0, s)

  @pl.when(c == 1)
  def _dump_q():
    _dump_acc(acc, q_out, 0, row0, s)


_E0 = 9920
_E1 = 10080
_SUBS0 = ((0, 4800), (4800, 4800), (9600, 320))
_SUBS1 = ((0, 4800), (4800, 4800), (9600, 480))


@functools.partial(
    pl.kernel,
    out_type=jax.ShapeDtypeStruct((NC * N, D), jnp.float32),
    mesh=_SC_MESH,
    scratch_types=[
        pltpu.VMEM((4800,), jnp.int32),
        pltpu.VMEM((_K2,), jnp.int32),
        pltpu.VMEM((_K2,), jnp.int32),
        pltpu.VMEM((_K2, D), jnp.float32),
        pltpu.VMEM((_K2, D), jnp.float32),
        pltpu.VMEM_SHARED((N, D), jnp.float32),
        pltpu.SemaphoreType.DMA,
        pltpu.SemaphoreType.DMA,
    ])
def _agg1(table_hbm, src_hbm, dst_hbm, zeros_hbm, out,
          idx_s, idx_da, idx_db, rows_a, rows_b, acc, sem_ga, sem_gb):
  c = lax.axis_index("c")
  s = lax.axis_index("s")
  row0 = s * RPT

  _zero_acc(zeros_hbm, acc, row0, s)
  plsc.subcore_barrier()

  def run(ebase, subs):
    for off, ln in subs:
      pltpu.sync_copy(src_hbm.at[pl.ds(ebase + off, ln)],
                      idx_s.at[pl.ds(0, ln)])
      _make_pipeline(dst_hbm, idx_s, acc,
                     (idx_da, idx_db, rows_a, rows_b, sem_ga, sem_gb),
                     ebase + off, ln // _K2, k=_K2)(table_hbm)

  @pl.when(c == 0)
  def _lo():
    run(s * _E0, _SUBS0)

  @pl.when(c == 1)
  def _hi():
    run(NS * _E0 + s * _E1, _SUBS1)

  plsc.subcore_barrier()
  _dump_acc(acc, out, c * N, row0, s)


_R = 2000


def _row_spec(off_blocks, width=D):
  return pl.BlockSpec((_R, width), lambda i, o=off_blocks: (o + i, 0))


def _full_spec(shape):
  return pl.BlockSpec(shape, lambda i: (0, 0))


def _gate1_body(p, q, f, st, wnt, wnb, wst, wsb, b, z_o, rs_o):
  acc = jnp.dot(p[...], wnt[...], preferred_element_type=jnp.float32)
  acc += jnp.dot(q[...], wnb[...], preferred_element_type=jnp.float32)
  acc += jnp.dot(f[...], wst[...], preferred_element_type=jnp.float32)
  acc += jnp.dot(st[...], wsb[...], preferred_element_type=jnp.float32)
  zr = jax.nn.sigmoid(acc + b[...])
  z_o[...] = zr[:, :D]
  rs_o[...] = zr[:, D:] * st[...]


def _half_spec(half, width):
  return pl.BlockSpec((D, width), lambda i, h=half: (h, 0))


def _gate1(p, q, feat, state, w_nbr, w_self, b):
  in_specs = [
      _row_spec(0), _row_spec(0),
      _row_spec(0), _row_spec(0),
      _half_spec(0, 2 * D), _half_spec(1, 2 * D),
      _half_spec(0, 2 * D), _half_spec(1, 2 * D),
      _full_spec((1, 2 * D)),
  ]
  return pl.pallas_call(
      _gate1_body, grid=(N // _R,), in_specs=in_specs,
      out_specs=[_row_spec(0)] * 2,
      out_shape=[jax.ShapeDtypeStruct((N, D), jnp.float32)] * 2,
  )(p, q, feat, state, w_nbr, w_nbr, w_self, w_self, b)


def _gate2_body(p, s0, s1, f, rs, z, st, wnt, wnb, wst, wsb, b, out):
  acc = jnp.dot(p[...], wnt[...], preferred_element_type=jnp.float32)
  acc += jnp.dot(s0[...] + s1[...], wnb[...],
                 preferred_element_type=jnp.float32)
  acc += jnp.dot(f[...], wst[...], preferred_element_type=jnp.float32)
  acc += jnp.dot(rs[...], wsb[...], preferred_element_type=jnp.float32)
  c = jnp.tanh(acc + b[...])
  zz = z[...]
  out[...] = zz * st[...] + (1.0 - zz) * c


def _gate2(p, s2, feat, rs, z, state, w_nbr, w_self, b):
  nb = N // _R
  in_specs = [
      _row_spec(0),
      _row_spec(0), _row_spec(nb),
      _row_spec(0), _row_spec(0), _row_spec(0), _row_spec(0),
      _half_spec(0, D), _half_spec(1, D),
      _half_spec(0, D), _half_spec(1, D),
      _full_spec((1, D)),
  ]
  return pl.pallas_call(
      _gate2_body, grid=(nb,), in_specs=in_specs,
      out_specs=_row_spec(0),
      out_shape=jax.ShapeDtypeStruct((N, D), jnp.float32),
  )(p, s2, s2, feat, rs, z, state, w_nbr, w_nbr, w_self, w_self, b)


def kernel(feat, state, edge_index, W_zr_nbr, W_zr_self, b_zr,
           W_c_nbr, W_c_self, b_c):
  src = edge_index[0].astype(jnp.int32)
  dst = edge_index[1].astype(jnp.int32)
  zeros = jnp.zeros((RPT, D), jnp.float32)

  p, q = _agg2(feat, state, src, dst, zeros)
  z, rs = _gate1(p, q, feat, state, W_zr_nbr, W_zr_self,
                 b_zr.reshape(1, 2 * D))
  s2 = _agg1(rs, src, dst, zeros)
  return _gate2(p, s2, feat, rs, z, state, W_c_nbr, W_c_self,
                b_c.reshape(1, D))
